# baseline (speedup 1.0000x reference)
import jax
import jax.numpy as jnp
from jax import lax
from jax.experimental import pallas as pl
from jax.experimental.pallas import tpu as pltpu
from jax.experimental.pallas import tpu_sc as plsc

N = 10000
E = 320000
IN_FEATS = 128
H = 128
C = 64
F = 5

NC = 2
NS = 16
LANE = 128
CH = 160
CH2 = 160
LANEH = 64
E_PAD = NS * CH * LANE
N_PAD = 10112
NROWCH = N_PAD // 128
MAXT = 5
SLB = 16
SLBH = 32

_ROWS = (0, 0, 0, 0, 1, 1, 1, 2, 2, 3)
_COLS = (1, 2, 3, 4, 2, 3, 4, 3, 4, 4)

f32 = jnp.float32
i32 = jnp.int32


def _make_sc_mesh():
    return plsc.VectorSubcoreMesh(
        core_axis_name="c", subcore_axis_name="s", num_cores=NC, num_subcores=NS
    )


def _deg_body(dstp, ones1, zdeg, out, dst_slab, ones_v, dsem, deg_sp):
    c = lax.axis_index("c")
    s = lax.axis_index("s")
    pltpu.sync_copy(ones1, ones_v)
    for t in range(MAXT):
        ci = s + NS * t

        @pl.when(ci < NROWCH)
        def _zero():
            sl = pl.ds(ci * 128, 128)
            pltpu.sync_copy(zdeg.at[sl], deg_sp.at[sl])

    plsc.subcore_barrier()

    def group(gi, carry):
        pltpu.sync_copy(dstp.at[s, pl.ds(gi * SLB, SLB)], dst_slab)
        for j in range(SLB):
            pltpu.async_copy(ones_v, deg_sp.at[dst_slab.at[j]], dsem, add=True)
        for j in range(SLB):
            pltpu.make_async_copy(ones_v, deg_sp.at[dst_slab.at[0]], dsem).wait()
        return carry

    lax.fori_loop(0, CH // SLB, group, 0)
    plsc.subcore_barrier()
    for t in range(MAXT):
        ci = s + NS * t

        @pl.when(ci < NROWCH)
        def _write():
            sl = pl.ds(ci * 128, 128)
            pltpu.sync_copy(deg_sp.at[sl], out.at[c, sl])


NBUF = 5
LOOKAHEAD = NBUF - 1


def _hop_body(srcp, dstp, xin, zrow, out,
              src_slab, dst_slab, b0, b1, b2, b3, b4,
              g0, g1, g2, g3, g4, s0, s1, s2, s3, s4, acc):
    c = lax.axis_index("c")
    s = lax.axis_index("s")
    bufs = (b0, b1, b2, b3, b4)
    gsems = (g0, g1, g2, g3, g4)
    ssems = (s0, s1, s2, s3, s4)
    for t in range(MAXT):
        ci = s + NS * t

        @pl.when(ci < NROWCH)
        def _stage():
            sl = pl.ds(ci * 128, 128)
            pltpu.sync_copy(zrow.at[sl], acc.at[sl])

    plsc.subcore_barrier()

    def group(gi, carry):
        pltpu.sync_copy(srcp.at[c, s, pl.ds(gi * SLBH, SLBH)], src_slab)
        pltpu.sync_copy(dstp.at[c, s, pl.ds(gi * SLBH, SLBH)], dst_slab)
        for j in range(LOOKAHEAD):
            pltpu.async_copy(xin.at[src_slab.at[j]], bufs[j], gsems[j])
        for j in range(SLBH):
            b = j % NBUF
            pltpu.make_async_copy(
                xin.at[src_slab.at[0]], bufs[b], gsems[b]).wait()
            pltpu.async_copy(bufs[b], acc.at[dst_slab.at[j]], ssems[b],
                             add=True)
            nxt = j + LOOKAHEAD
            if nxt < SLBH:
                nb = nxt % NBUF
                if nxt >= NBUF:
                    pltpu.make_async_copy(
                        bufs[nb], acc.at[dst_slab.at[0]], ssems[nb]).wait()
                pltpu.async_copy(xin.at[src_slab.at[nxt]], bufs[nb], gsems[nb])
        for j in range(SLBH - NBUF, SLBH):
            b = j % NBUF
            pltpu.make_async_copy(
                bufs[b], acc.at[dst_slab.at[0]], ssems[b]).wait()
        return carry

    lax.fori_loop(0, CH2 // SLBH, group, 0)
    plsc.subcore_barrier()
    for t in range(MAXT):
        ci = s + NS * t

        @pl.when(ci < NROWCH)
        def _write():
            sl = pl.ds(ci * 128, 128)
            pltpu.sync_copy(acc.at[sl], out.at[c, sl])


def _build_deg_kernel():
    return pl.kernel(
        _deg_body,
        out_type=jax.ShapeDtypeStruct((NC, N_PAD), f32),
        mesh=_make_sc_mesh(),
        scratch_types=[
            pltpu.VMEM((SLB, LANE), i32),
            pltpu.VMEM((LANE,), f32),
            pltpu.SemaphoreType.DMA,
            pltpu.VMEM_SHARED((N_PAD,), f32),
        ],
    )


def _build_hop_kernel():
    return pl.kernel(
        _hop_body,
        out_type=jax.ShapeDtypeStruct((NC, N_PAD, H), f32),
        mesh=_make_sc_mesh(),
        scratch_types=(
            [pltpu.VMEM((SLBH, LANEH), i32)] * 2
            + [pltpu.VMEM((LANEH, H), f32)] * NBUF
            + [pltpu.SemaphoreType.DMA] * (2 * NBUF)
            + [pltpu.VMEM_SHARED((N_PAD, H), f32)]
        ),
    )


BN_EW = 1000
BN_MUL = 128
BN_FIN = 400


def _scale_body(deg_ref, x_ref, xn_ref, norm_ref):
    dc = jnp.maximum(deg_ref[...], 1.0)
    nrm = lax.rsqrt(dc)
    xn_ref[...] = x_ref[...] * nrm
    norm_ref[...] = nrm


def _comb_body(a_ref, n_ref, out_ref):
    nn = n_ref[...]
    out_ref[...] = (a_ref[0] + a_ref[1]) * (nn * nn)


def _final_body(a_ref, n_ref, idx_ref, val_ref, wsg_ref, bsg_ref, embp_ref,
                watt_ref, batt_ref, wft_ref, wfb_ref, bfin_ref, out_ref):
    hrow = (a_ref[0] + a_ref[1]) * n_ref[...]
    gnn = jnp.dot(hrow, wsg_ref[...], preferred_element_type=f32) + bsg_ref[...]

    iota = lax.broadcasted_iota(i32, (BN_FIN, 2 * H), 1)
    embp = embp_ref[...]
    xf = []
    for fld in range(F):
        onehot = (iota == idx_ref[:, fld:fld + 1]).astype(f32)
        row = jnp.dot(onehot, embp, preferred_element_type=f32)
        xf.append(row * val_ref[:, fld:fld + 1])

    watt = watt_ref[...]
    batt = batt_ref[...]
    ips, scores = [], []
    for i, j in zip(_ROWS, _COLS):
        ip = xf[i] * xf[j]
        fm = jnp.maximum(jnp.dot(ip, watt, preferred_element_type=f32) + batt, 0.0)
        scores.append(jnp.sum(gnn * fm, axis=1, keepdims=True))
        ips.append(ip)

    m = scores[0]
    for sc in scores[1:]:
        m = jnp.maximum(m, sc)
    es = [jnp.exp(sc - m) for sc in scores]
    den = es[0]
    for e in es[1:]:
        den = den + e
    attn = es[0] * ips[0]
    for e, ip in zip(es[1:], ips[1:]):
        attn = attn + e * ip
    attn = attn / den * 100.0

    out_ref[...] = (jnp.dot(gnn, wft_ref[...], preferred_element_type=f32)
                    + jnp.dot(attn, wfb_ref[...], preferred_element_type=f32)
                    + bfin_ref[...])


def _full(shape):
    return pl.BlockSpec(shape, lambda i: (0,) * len(shape))


def _scale_call(deg2, x):
    grid = N // BN_EW
    return pl.pallas_call(
        _scale_body,
        grid=(grid,),
        in_specs=[
            pl.BlockSpec((BN_EW, 1), lambda i: (i, 0)),
            pl.BlockSpec((BN_EW, H), lambda i: (i, 0)),
        ],
        out_specs=[
            pl.BlockSpec((BN_EW, H), lambda i: (i, 0)),
            pl.BlockSpec((BN_EW, 1), lambda i: (i, 0)),
        ],
        out_shape=[
            jax.ShapeDtypeStruct((N_PAD, H), f32),
            jax.ShapeDtypeStruct((N_PAD, 1), f32),
        ],
    )(deg2, x)


def _comb_call(a1, norm):
    grid = (N_PAD // BN_MUL,)
    return pl.pallas_call(
        _comb_body,
        grid=grid,
        in_specs=[
            pl.BlockSpec((NC, BN_MUL, H), lambda i: (0, i, 0)),
            pl.BlockSpec((BN_MUL, 1), lambda i: (i, 0)),
        ],
        out_specs=pl.BlockSpec((BN_MUL, H), lambda i: (i, 0)),
        out_shape=jax.ShapeDtypeStruct((N_PAD, H), f32),
    )(a1, norm)


def _final_call(a2, norm, nz_idx, nz_val, W_sg, b_sg, embp, W_att, b_att,
                W_ft, W_fb, b_fin):
    grid = N // BN_FIN
    return pl.pallas_call(
        _final_body,
        grid=(grid,),
        in_specs=[
            pl.BlockSpec((NC, BN_FIN, H), lambda i: (0, i, 0)),
            pl.BlockSpec((BN_FIN, 1), lambda i: (i, 0)),
            pl.BlockSpec((BN_FIN, F), lambda i: (i, 0)),
            pl.BlockSpec((BN_FIN, F), lambda i: (i, 0)),
            _full((H, H)),
            _full((1, H)),
            _full((2 * H, H)),
            _full((H, H)),
            _full((1, H)),
            _full((H, C)),
            _full((H, C)),
            _full((1, C)),
        ],
        out_specs=pl.BlockSpec((BN_FIN, C), lambda i: (i, 0)),
        out_shape=jax.ShapeDtypeStruct((N, C), f32),
    )(a2, norm, nz_idx, nz_val, W_sg, b_sg, embp, W_att, b_att, W_ft, W_fb,
      b_fin)


def kernel(x, edge_index, nonzero_index, nonzero_value, W_sg, b_sg, emb,
           W_att, b_att, W_fin, b_fin):
    src = edge_index[0]
    dst = edge_index[1]
    pad = E_PAD - E
    srcf = jnp.concatenate([src, jnp.zeros((pad,), i32)])
    dstf = jnp.concatenate([dst, jnp.full((pad,), N, i32)])
    dstp_deg = dstf.reshape(NS, CH, LANE)
    srcp = srcf.reshape(NC, NS, CH2, LANEH)
    dstp = dstf.reshape(NC, NS, CH2, LANEH)
    ones1 = jnp.ones((LANE,), f32)
    zdeg = jnp.zeros((N_PAD,), f32)
    zrow = jnp.zeros((N_PAD, H), f32)
    embp = jnp.concatenate([emb, jnp.zeros((2 * H - (IN_FEATS + 1), H), f32)], 0)

    deg2 = _build_deg_kernel()(dstp_deg, ones1, zdeg)
    degcol = deg2[0, :N][:, None]
    xns, norm = _scale_call(degcol, x)
    hop = _build_hop_kernel()
    a1 = hop(srcp, dstp, xns, zrow)
    h1 = _comb_call(a1, norm)
    a2 = hop(srcp, dstp, h1, zrow)

    return _final_call(
        a2, norm, nonzero_index, nonzero_value,
        W_sg, b_sg.reshape(1, H), embp, W_att, b_att.reshape(1, H),
        W_fin[:H], W_fin[H:], b_fin.reshape(1, C),
    )


# speedup vs baseline: 1.0064x; 1.0064x over previous
import jax
import jax.numpy as jnp
from jax import lax
from jax.experimental import pallas as pl
from jax.experimental.pallas import tpu as pltpu
from jax.experimental.pallas import tpu_sc as plsc

N = 10000
E = 320000
IN_FEATS = 128
H = 128
C = 64
F = 5

NC = 2
NS = 16
LANE = 128
CH = 160
CH_D = 80
CH2 = 160
LANEH = 64
E_PAD = NS * CH * LANE
N_PAD = 10112
NROWCH = N_PAD // 128
MAXT = 5
SLB = 16
SLBH = 32

_ROWS = (0, 0, 0, 0, 1, 1, 1, 2, 2, 3)
_COLS = (1, 2, 3, 4, 2, 3, 4, 3, 4, 4)

f32 = jnp.float32
i32 = jnp.int32


def _make_sc_mesh():
    return plsc.VectorSubcoreMesh(
        core_axis_name="c", subcore_axis_name="s", num_cores=NC, num_subcores=NS
    )


def _deg_body(dstp, ones1, zdeg, out, dst_slab, ones_v, dsem, deg_sp):
    c = lax.axis_index("c")
    s = lax.axis_index("s")
    pltpu.sync_copy(ones1, ones_v)
    for t in range(MAXT):
        ci = s + NS * t

        @pl.when(ci < NROWCH)
        def _zero():
            sl = pl.ds(ci * 128, 128)
            pltpu.sync_copy(zdeg.at[sl], deg_sp.at[sl])

    plsc.subcore_barrier()

    def group(gi, carry):
        pltpu.sync_copy(dstp.at[c, s, pl.ds(gi * SLB, SLB)], dst_slab)
        for j in range(SLB):
            pltpu.async_copy(ones_v, deg_sp.at[dst_slab.at[j]], dsem, add=True)
        for j in range(SLB):
            pltpu.make_async_copy(ones_v, deg_sp.at[dst_slab.at[0]], dsem).wait()
        return carry

    lax.fori_loop(0, CH_D // SLB, group, 0)
    plsc.subcore_barrier()
    for t in range(MAXT):
        ci = s + NS * t

        @pl.when(ci < NROWCH)
        def _write():
            sl = pl.ds(ci * 128, 128)
            pltpu.sync_copy(deg_sp.at[sl], out.at[c, sl])


NBUF = 5
LOOKAHEAD = NBUF - 1


def _hop_body(srcp, dstp, xin, zrow, out,
              src_slab, dst_slab, b0, b1, b2, b3, b4,
              g0, g1, g2, g3, g4, s0, s1, s2, s3, s4, acc):
    c = lax.axis_index("c")
    s = lax.axis_index("s")
    bufs = (b0, b1, b2, b3, b4)
    gsems = (g0, g1, g2, g3, g4)
    ssems = (s0, s1, s2, s3, s4)
    for t in range(MAXT):
        ci = s + NS * t

        @pl.when(ci < NROWCH)
        def _stage():
            sl = pl.ds(ci * 128, 128)
            pltpu.sync_copy(zrow.at[sl], acc.at[sl])

    plsc.subcore_barrier()

    def group(gi, carry):
        pltpu.sync_copy(srcp.at[c, s, pl.ds(gi * SLBH, SLBH)], src_slab)
        pltpu.sync_copy(dstp.at[c, s, pl.ds(gi * SLBH, SLBH)], dst_slab)
        for j in range(LOOKAHEAD):
            pltpu.async_copy(xin.at[src_slab.at[j]], bufs[j], gsems[j])
        for j in range(SLBH):
            b = j % NBUF
            pltpu.make_async_copy(
                xin.at[src_slab.at[0]], bufs[b], gsems[b]).wait()
            pltpu.async_copy(bufs[b], acc.at[dst_slab.at[j]], ssems[b],
                             add=True)
            nxt = j + LOOKAHEAD
            if nxt < SLBH:
                nb = nxt % NBUF
                if nxt >= NBUF:
                    pltpu.make_async_copy(
                        bufs[nb], acc.at[dst_slab.at[0]], ssems[nb]).wait()
                pltpu.async_copy(xin.at[src_slab.at[nxt]], bufs[nb], gsems[nb])
        for j in range(SLBH - NBUF, SLBH):
            b = j % NBUF
            pltpu.make_async_copy(
                bufs[b], acc.at[dst_slab.at[0]], ssems[b]).wait()
        return carry

    lax.fori_loop(0, CH2 // SLBH, group, 0)
    plsc.subcore_barrier()
    for t in range(MAXT):
        ci = s + NS * t

        @pl.when(ci < NROWCH)
        def _write():
            sl = pl.ds(ci * 128, 128)
            pltpu.sync_copy(acc.at[sl], out.at[c, sl])


def _build_deg_kernel():
    return pl.kernel(
        _deg_body,
        out_type=jax.ShapeDtypeStruct((NC, N_PAD), f32),
        mesh=_make_sc_mesh(),
        scratch_types=[
            pltpu.VMEM((SLB, LANE), i32),
            pltpu.VMEM((LANE,), f32),
            pltpu.SemaphoreType.DMA,
            pltpu.VMEM_SHARED((N_PAD,), f32),
        ],
    )


def _build_hop_kernel():
    return pl.kernel(
        _hop_body,
        out_type=jax.ShapeDtypeStruct((NC, N_PAD, H), f32),
        mesh=_make_sc_mesh(),
        scratch_types=(
            [pltpu.VMEM((SLBH, LANEH), i32)] * 2
            + [pltpu.VMEM((LANEH, H), f32)] * NBUF
            + [pltpu.SemaphoreType.DMA] * (2 * NBUF)
            + [pltpu.VMEM_SHARED((N_PAD, H), f32)]
        ),
    )


BN_EW = 1000
BN_MUL = 128
BN_FIN = 400


def _scale_body(deg_ref, x_ref, xn_ref, norm_ref):
    dc = jnp.maximum(deg_ref[...], 1.0)
    nrm = lax.rsqrt(dc)
    xn_ref[...] = x_ref[...] * nrm
    norm_ref[...] = nrm


def _comb_body(a_ref, n_ref, out_ref):
    nn = n_ref[...]
    out_ref[...] = (a_ref[0] + a_ref[1]) * (nn * nn)


def _final_body(a_ref, n_ref, idx_ref, val_ref, wsg_ref, bsg_ref, embp_ref,
                watt_ref, batt_ref, wft_ref, wfb_ref, bfin_ref, out_ref):
    hrow = (a_ref[0] + a_ref[1]) * n_ref[...]
    gnn = jnp.dot(hrow, wsg_ref[...], preferred_element_type=f32) + bsg_ref[...]

    iota = lax.broadcasted_iota(i32, (BN_FIN, 2 * H), 1)
    embp = embp_ref[...]
    xf = []
    for fld in range(F):
        onehot = (iota == idx_ref[:, fld:fld + 1]).astype(f32)
        row = jnp.dot(onehot, embp, preferred_element_type=f32)
        xf.append(row * val_ref[:, fld:fld + 1])

    watt = watt_ref[...]
    batt = batt_ref[...]
    ips, scores = [], []
    for i, j in zip(_ROWS, _COLS):
        ip = xf[i] * xf[j]
        fm = jnp.maximum(jnp.dot(ip, watt, preferred_element_type=f32) + batt, 0.0)
        scores.append(jnp.sum(gnn * fm, axis=1, keepdims=True))
        ips.append(ip)

    m = scores[0]
    for sc in scores[1:]:
        m = jnp.maximum(m, sc)
    es = [jnp.exp(sc - m) for sc in scores]
    den = es[0]
    for e in es[1:]:
        den = den + e
    attn = es[0] * ips[0]
    for e, ip in zip(es[1:], ips[1:]):
        attn = attn + e * ip
    attn = attn / den * 100.0

    out_ref[...] = (jnp.dot(gnn, wft_ref[...], preferred_element_type=f32)
                    + jnp.dot(attn, wfb_ref[...], preferred_element_type=f32)
                    + bfin_ref[...])


def _full(shape):
    return pl.BlockSpec(shape, lambda i: (0,) * len(shape))


def _scale_call(deg2, x):
    grid = N // BN_EW
    return pl.pallas_call(
        _scale_body,
        grid=(grid,),
        in_specs=[
            pl.BlockSpec((BN_EW, 1), lambda i: (i, 0)),
            pl.BlockSpec((BN_EW, H), lambda i: (i, 0)),
        ],
        out_specs=[
            pl.BlockSpec((BN_EW, H), lambda i: (i, 0)),
            pl.BlockSpec((BN_EW, 1), lambda i: (i, 0)),
        ],
        out_shape=[
            jax.ShapeDtypeStruct((N_PAD, H), f32),
            jax.ShapeDtypeStruct((N_PAD, 1), f32),
        ],
    )(deg2, x)


def _comb_call(a1, norm):
    grid = (N_PAD // BN_MUL,)
    return pl.pallas_call(
        _comb_body,
        grid=grid,
        in_specs=[
            pl.BlockSpec((NC, BN_MUL, H), lambda i: (0, i, 0)),
            pl.BlockSpec((BN_MUL, 1), lambda i: (i, 0)),
        ],
        out_specs=pl.BlockSpec((BN_MUL, H), lambda i: (i, 0)),
        out_shape=jax.ShapeDtypeStruct((N_PAD, H), f32),
    )(a1, norm)


def _final_call(a2, norm, nz_idx, nz_val, W_sg, b_sg, embp, W_att, b_att,
                W_ft, W_fb, b_fin):
    grid = N // BN_FIN
    return pl.pallas_call(
        _final_body,
        grid=(grid,),
        in_specs=[
            pl.BlockSpec((NC, BN_FIN, H), lambda i: (0, i, 0)),
            pl.BlockSpec((BN_FIN, 1), lambda i: (i, 0)),
            pl.BlockSpec((BN_FIN, F), lambda i: (i, 0)),
            pl.BlockSpec((BN_FIN, F), lambda i: (i, 0)),
            _full((H, H)),
            _full((1, H)),
            _full((2 * H, H)),
            _full((H, H)),
            _full((1, H)),
            _full((H, C)),
            _full((H, C)),
            _full((1, C)),
        ],
        out_specs=pl.BlockSpec((BN_FIN, C), lambda i: (i, 0)),
        out_shape=jax.ShapeDtypeStruct((N, C), f32),
    )(a2, norm, nz_idx, nz_val, W_sg, b_sg, embp, W_att, b_att, W_ft, W_fb,
      b_fin)


def kernel(x, edge_index, nonzero_index, nonzero_value, W_sg, b_sg, emb,
           W_att, b_att, W_fin, b_fin):
    src = edge_index[0]
    dst = edge_index[1]
    pad = E_PAD - E
    srcf = jnp.concatenate([src, jnp.zeros((pad,), i32)])
    dstf = jnp.concatenate([dst, jnp.full((pad,), N, i32)])
    dstp_deg = dstf.reshape(NC, NS, CH_D, LANE)
    srcp = srcf.reshape(NC, NS, CH2, LANEH)
    dstp = dstf.reshape(NC, NS, CH2, LANEH)
    ones1 = jnp.ones((LANE,), f32)
    zdeg = jnp.zeros((N_PAD,), f32)
    zrow = jnp.zeros((N_PAD, H), f32)
    embp = jnp.concatenate([emb, jnp.zeros((2 * H - (IN_FEATS + 1), H), f32)], 0)

    deg2 = _build_deg_kernel()(dstp_deg, ones1, zdeg)
    degcol = (deg2[0, :N] + deg2[1, :N])[:, None]
    xns, norm = _scale_call(degcol, x)
    hop = _build_hop_kernel()
    a1 = hop(srcp, dstp, xns, zrow)
    h1 = _comb_call(a1, norm)
    a2 = hop(srcp, dstp, h1, zrow)

    return _final_call(
        a2, norm, nonzero_index, nonzero_value,
        W_sg, b_sg.reshape(1, H), embp, W_att, b_att.reshape(1, H),
        W_fin[:H], W_fin[H:], b_fin.reshape(1, C),
    )
